# Initial kernel scaffold; baseline (speedup 1.0000x reference)
import jax
import jax.numpy as jnp
from jax.experimental import pallas as pl


def kernel(node_features, mask, W, b, noise):
    raise NotImplementedError("write your pallas kernel here")


# speedup vs baseline: 1.0429x; 1.0429x over previous
import jax
import jax.numpy as jnp
from jax.experimental import pallas as pl

_INTERPRET = False

_RANK_CHUNK = 256


def _scores_body(nf_ref, noise_ref, w_ref, b_ref, s_ref, min_ref):
    x = nf_ref[0] + 0.05 * noise_ref[0]
    s = jnp.sum(x * w_ref[...], axis=1, keepdims=True)
    s = s + b_ref[0, 0]
    s_ref[0] = s
    min_ref[...] = jnp.min(s).reshape(1, 1, 1)


def _sort_body(scol_ref, srow_ref, minv_ref, mcol_ref, mrow_ref,
               smask_ref, ss_ref):
    n = scol_ref.shape[1]
    fill = jnp.min(minv_ref[...]) - 1.0
    scol = jnp.where(mcol_ref[0] != 0, scol_ref[0], fill)
    srow = jnp.where(mrow_ref[0] != 0, srow_ref[0], fill)
    smask_ref[0] = scol

    ch = _RANK_CHUNK
    acc = jnp.zeros((1, n), jnp.float32)
    col_iota = jax.lax.broadcasted_iota(jnp.int32, (ch, n), 1)
    for c in range(n // ch):
        sj = jax.lax.slice(scol, (c * ch, 0), ((c + 1) * ch, 1))
        row_iota = jax.lax.broadcasted_iota(jnp.int32, (ch, n), 0) + c * ch
        gt = srow > sj
        tie = (srow == sj) & (col_iota < row_iota)
        rank = jnp.sum((gt | tie).astype(jnp.int32), axis=1, keepdims=True)
        onehot = col_iota == rank
        acc = acc + jnp.sum(jnp.where(onehot, sj, 0.0), axis=0, keepdims=True)
    ss_ref[0] = acc


def _perm_body(s_ref, ss_ref, m_ref, out_ref):
    n, ibk = out_ref.shape[1], out_ref.shape[2]
    ib = pl.program_id(1)
    scol = s_ref[0]
    ssrow = ss_ref[0]
    c = (jnp.max(scol) + jnp.min(scol)) * 0.5
    u = jnp.exp(scol - c)
    ru = 1.0 / u
    v = jnp.exp(ssrow - c)
    rv = 1.0 / v
    e = jnp.minimum(u * rv, ru * v)
    denom = jnp.sum(e, axis=0, keepdims=True)
    p = e * (1.0 / denom)
    rows = jax.lax.broadcasted_iota(jnp.int32, (n, ibk), 0)
    cols = jax.lax.broadcasted_iota(jnp.int32, (n, ibk), 1) + ib * ibk
    eye = (rows == cols).astype(jnp.float32)
    out_ref[0] = jnp.where(m_ref[0] != 0, p, eye)


def kernel(node_features, mask, W, b, noise):
    B, N, D = node_features.shape
    mask_i = mask.astype(jnp.int32)
    w_row = W.reshape(1, D)
    b2 = b.reshape(1, 1)

    scores_col, minv = pl.pallas_call(
        _scores_body,
        grid=(B,),
        in_specs=[
            pl.BlockSpec((1, N, D), lambda i: (i, 0, 0)),
            pl.BlockSpec((1, N, D), lambda i: (i, 0, 0)),
            pl.BlockSpec((1, D), lambda i: (0, 0)),
            pl.BlockSpec((1, 1), lambda i: (0, 0)),
        ],
        out_specs=[
            pl.BlockSpec((1, N, 1), lambda i: (i, 0, 0)),
            pl.BlockSpec((1, 1, 1), lambda i: (i, 0, 0)),
        ],
        out_shape=[
            jax.ShapeDtypeStruct((B, N, 1), jnp.float32),
            jax.ShapeDtypeStruct((B, 1, 1), jnp.float32),
        ],
        interpret=_INTERPRET,
    )(node_features, noise, w_row, b2)

    scores_row = scores_col.reshape(B, 1, N)
    mask_col = mask_i.reshape(B, N, 1)
    mask_row = mask_i.reshape(B, 1, N)

    smask_col, ss_row = pl.pallas_call(
        _sort_body,
        grid=(B,),
        in_specs=[
            pl.BlockSpec((1, N, 1), lambda i: (i, 0, 0)),
            pl.BlockSpec((1, 1, N), lambda i: (i, 0, 0)),
            pl.BlockSpec((B, 1, 1), lambda i: (0, 0, 0)),
            pl.BlockSpec((1, N, 1), lambda i: (i, 0, 0)),
            pl.BlockSpec((1, 1, N), lambda i: (i, 0, 0)),
        ],
        out_specs=[
            pl.BlockSpec((1, N, 1), lambda i: (i, 0, 0)),
            pl.BlockSpec((1, 1, N), lambda i: (i, 0, 0)),
        ],
        out_shape=[
            jax.ShapeDtypeStruct((B, N, 1), jnp.float32),
            jax.ShapeDtypeStruct((B, 1, N), jnp.float32),
        ],
        interpret=_INTERPRET,
    )(scores_col, scores_row, minv, mask_col, mask_row)

    IBK = 512
    out = pl.pallas_call(
        _perm_body,
        grid=(B, N // IBK),
        in_specs=[
            pl.BlockSpec((1, N, 1), lambda bb, ib: (bb, 0, 0)),
            pl.BlockSpec((1, 1, IBK), lambda bb, ib: (bb, 0, ib)),
            pl.BlockSpec((1, N, 1), lambda bb, ib: (bb, 0, 0)),
        ],
        out_specs=pl.BlockSpec((1, N, IBK), lambda bb, ib: (bb, 0, ib)),
        out_shape=jax.ShapeDtypeStruct((B, N, N), jnp.float32),
        interpret=_INTERPRET,
    )(smask_col, ss_row, mask_col)
    return out
